# baseline (speedup 1.0000x reference)
import functools

import jax
import jax.numpy as jnp
from jax import lax
from jax.experimental import pallas as pl
from jax.experimental.pallas import tpu as pltpu
from jax.experimental.pallas import tpu_sc as plsc

_VOCAB = 100000
_MAX_POS = 2048
_D = 768
_BATCH = 4
_SEQ = 2048
_N = _BATCH * _SEQ
_NC = 2
_NS = 16
_NW = _NC * _NS
_PER_W = _N // _NW
_CHUNK = 64
_NCHUNK = _PER_W // _CHUNK
_LANES = 16
_VECS_PER_ROW = _D // _LANES


def _emb_body(ids_hbm, tok_hbm, pos_hbm, out_hbm, idx_v, tok_v, pos_v, sem):
    wid = lax.axis_index("s") * _NC + lax.axis_index("c")
    base = wid * _PER_W
    pos_base = lax.rem(base, _SEQ)

    pltpu.sync_copy(ids_hbm.at[pl.ds(base, _PER_W)], idx_v)

    for c in range(_NCHUNK):
        off = c * _CHUNK
        pltpu.async_copy(tok_hbm.at[idx_v.at[pl.ds(off, _CHUNK)]], tok_v,
                         sem).wait()
        pltpu.sync_copy(pos_hbm.at[pl.ds(pos_base + off, _CHUNK)], pos_v)

        def _row(r, carry):
            def _col(j, carry2):
                s = pl.ds(j * _LANES, _LANES)
                tok_v[r, s] = tok_v[r, s] + pos_v[r, s]
                return carry2
            return lax.fori_loop(0, _VECS_PER_ROW, _col, carry, unroll=8)
        lax.fori_loop(0, _CHUNK, _row, 0)

        pltpu.sync_copy(tok_v, out_hbm.at[pl.ds(base + off, _CHUNK)])


@jax.jit
def _emb(ids_flat, token_table, pos_table):
    mesh = plsc.VectorSubcoreMesh(core_axis_name="c", subcore_axis_name="s")
    run = functools.partial(
        pl.kernel,
        mesh=mesh,
        out_type=jax.ShapeDtypeStruct((_N, _D), jnp.float32),
        scratch_types=[
            pltpu.VMEM((_PER_W,), jnp.int32),
            pltpu.VMEM((_CHUNK, _D), jnp.float32),
            pltpu.VMEM((_CHUNK, _D), jnp.float32),
            pltpu.SemaphoreType.DMA,
        ],
    )(_emb_body)
    return run(ids_flat, token_table, pos_table)


def kernel(input_ids, token_table, pos_table):
    ids_flat = input_ids.reshape(_N).astype(jnp.int32)
    out = _emb(ids_flat, token_table, pos_table)
    return out.reshape(_BATCH, _SEQ, _D)


# speedup vs baseline: 1.2021x; 1.2021x over previous
import functools

import jax
import jax.numpy as jnp
from jax import lax
from jax.experimental import pallas as pl
from jax.experimental.pallas import tpu as pltpu
from jax.experimental.pallas import tpu_sc as plsc

_VOCAB = 100000
_D = 768
_BATCH = 4
_SEQ = 2048
_N = _BATCH * _SEQ
_NC = 2
_NS = 16
_NW = _NC * _NS
_POS_W = _SEQ // _NW
_PSUB = 16
_NSUB = _POS_W // _PSUB
_ROWS = _BATCH * _PSUB
_LANES = 16
_VECS = _D // _LANES


def _emb_body(ids_hbm, tok_hbm, pos_hbm, out_hbm,
              ids_stage, idx_v, tv0, tv1, pv0, pv1,
              s_in0, s_in1, s_out0, s_out1):
    wid = lax.axis_index("s") * _NC + lax.axis_index("c")
    pos0 = wid * _POS_W

    tv = (tv0, tv1)
    pv = (pv0, pv1)
    s_in = (s_in0, s_in1)
    s_out = (s_out0, s_out1)

    for bat in range(_BATCH):
        pltpu.sync_copy(ids_hbm.at[pl.ds(bat * _SEQ + pos0, _POS_W)],
                        ids_stage.at[pl.ds(bat * _POS_W, _POS_W)])
    for h in range(_NSUB):
        for bat in range(_BATCH):
            idx_v[pl.ds((h * _BATCH + bat) * _PSUB, _PSUB)] = (
                ids_stage[pl.ds(bat * _POS_W + h * _PSUB, _PSUB)])

    def fire(h, b):
        g = pltpu.async_copy(tok_hbm.at[idx_v.at[pl.ds(h * _ROWS, _ROWS)]],
                             tv[b], s_in[b])
        p = pltpu.async_copy(pos_hbm.at[pl.ds(pos0 + h * _PSUB, _PSUB)],
                             pv[b], s_in[b])
        return (g, p)

    in_h = [None, None]
    out_h = [[], []]
    in_h[0] = fire(0, 0)

    for h in range(_NSUB):
        b = h & 1
        nb = 1 - b
        if h + 1 < _NSUB:
            for hnd in out_h[nb]:
                hnd.wait()
            out_h[nb] = []
            in_h[nb] = fire(h + 1, nb)
        in_h[b][0].wait()
        in_h[b][1].wait()

        tvb, pvb = tv[b], pv[b]

        def _p(p, carry):
            def _j(j, carry2):
                s = pl.ds(j * _LANES, _LANES)
                pvec = pvb[p, s]
                for bat in range(_BATCH):
                    r = bat * _PSUB + p
                    tvb[r, s] = tvb[r, s] + pvec
                return carry2
            return lax.fori_loop(0, _VECS, _j, carry, unroll=4)
        lax.fori_loop(0, _PSUB, _p, 0)

        for bat in range(_BATCH):
            out_h[b].append(pltpu.async_copy(
                tvb.at[pl.ds(bat * _PSUB, _PSUB)],
                out_hbm.at[pl.ds(bat * _SEQ + pos0 + h * _PSUB, _PSUB)],
                s_out[b]))

    for hnds in out_h:
        for hnd in hnds:
            hnd.wait()


@jax.jit
def _emb(ids_flat, token_table, pos_table):
    mesh = plsc.VectorSubcoreMesh(core_axis_name="c", subcore_axis_name="s")
    run = functools.partial(
        pl.kernel,
        mesh=mesh,
        out_type=jax.ShapeDtypeStruct((_N, _D), jnp.float32),
        scratch_types=[
            pltpu.VMEM((_BATCH * _POS_W,), jnp.int32),
            pltpu.VMEM((_BATCH * _POS_W,), jnp.int32),
            pltpu.VMEM((_ROWS, _D), jnp.float32),
            pltpu.VMEM((_ROWS, _D), jnp.float32),
            pltpu.VMEM((_PSUB, _D), jnp.float32),
            pltpu.VMEM((_PSUB, _D), jnp.float32),
            pltpu.SemaphoreType.DMA,
            pltpu.SemaphoreType.DMA,
            pltpu.SemaphoreType.DMA,
            pltpu.SemaphoreType.DMA,
        ],
    )(_emb_body)
    return run(ids_flat, token_table, pos_table)


def kernel(input_ids, token_table, pos_table):
    ids_flat = input_ids.reshape(_N).astype(jnp.int32)
    out = _emb(ids_flat, token_table, pos_table)
    return out.reshape(_BATCH, _SEQ, _D)


# speedup vs baseline: 2.2443x; 1.8669x over previous
import functools

import jax
import jax.numpy as jnp
from jax import lax
from jax.experimental import pallas as pl
from jax.experimental.pallas import tpu as pltpu
from jax.experimental.pallas import tpu_sc as plsc

_VOCAB = 100000
_D = 768
_BATCH = 4
_SEQ = 2048
_N = _BATCH * _SEQ
_NC = 2
_NS = 16
_NW = _NC * _NS
_POS_W = _SEQ // _NW
_PSUB = 16
_NSUB = _POS_W // _PSUB
_ROWS = _BATCH * _PSUB
_LANES = 16
_VECS = _D // _LANES


def _emb_body(ids_hbm, tok_hbm, pos_hbm, out_hbm,
              ids_stage, idx_v, tv0, tv1, pv0, pv1,
              s_in0, s_in1, s_out0, s_out1):
    wid = lax.axis_index("s") * _NC + lax.axis_index("c")
    pos0 = wid * _POS_W

    tv = (tv0, tv1)
    pv = (pv0, pv1)
    s_in = (s_in0, s_in1)
    s_out = (s_out0, s_out1)

    for bat in range(_BATCH):
        pltpu.sync_copy(ids_hbm.at[pl.ds(bat * _SEQ + pos0, _POS_W)],
                        ids_stage.at[pl.ds(bat * _POS_W, _POS_W)])
    for h in range(_NSUB):
        for bat in range(_BATCH):
            idx_v[pl.ds((h * _BATCH + bat) * _PSUB, _PSUB)] = (
                ids_stage[pl.ds(bat * _POS_W + h * _PSUB, _PSUB)])

    def fire(h, b):
        g = pltpu.async_copy(tok_hbm.at[idx_v.at[pl.ds(h * _ROWS, _ROWS)]],
                             tv[b], s_in[b])
        p = pltpu.async_copy(pos_hbm.at[pl.ds(pos0 + h * _PSUB, _PSUB)],
                             pv[b], s_in[b])
        return (g, p)

    in_h = [None, None]
    out_h = [[], []]
    in_h[0] = fire(0, 0)

    for h in range(_NSUB):
        b = h & 1
        nb = 1 - b
        if h + 1 < _NSUB:
            for hnd in out_h[nb]:
                hnd.wait()
            out_h[nb] = []
            in_h[nb] = fire(h + 1, nb)
        in_h[b][0].wait()
        in_h[b][1].wait()

        tvb, pvb = tv[b], pv[b]

        @plsc.parallel_loop(0, _PSUB * _VECS, 1, unroll=4)
        def _add(k):
            p = k & (_PSUB - 1)
            j = k >> 4
            s = pl.ds(j * _LANES, _LANES)
            pvec = pvb[p, s]
            for bat in range(_BATCH):
                r = bat * _PSUB + p
                tvb[r, s] = tvb[r, s] + pvec

        for bat in range(_BATCH):
            out_h[b].append(pltpu.async_copy(
                tvb.at[pl.ds(bat * _PSUB, _PSUB)],
                out_hbm.at[pl.ds(bat * _SEQ + pos0 + h * _PSUB, _PSUB)],
                s_out[b]))

    for hnds in out_h:
        for hnd in hnds:
            hnd.wait()


@jax.jit
def _emb(ids_flat, token_table, pos_table):
    mesh = plsc.VectorSubcoreMesh(core_axis_name="c", subcore_axis_name="s")
    run = functools.partial(
        pl.kernel,
        mesh=mesh,
        out_type=jax.ShapeDtypeStruct((_N, _D), jnp.float32),
        scratch_types=[
            pltpu.VMEM((_BATCH * _POS_W,), jnp.int32),
            pltpu.VMEM((_BATCH * _POS_W,), jnp.int32),
            pltpu.VMEM((_ROWS, _D), jnp.float32),
            pltpu.VMEM((_ROWS, _D), jnp.float32),
            pltpu.VMEM((_PSUB, _D), jnp.float32),
            pltpu.VMEM((_PSUB, _D), jnp.float32),
            pltpu.SemaphoreType.DMA,
            pltpu.SemaphoreType.DMA,
            pltpu.SemaphoreType.DMA,
            pltpu.SemaphoreType.DMA,
        ],
    )(_emb_body)
    return run(ids_flat, token_table, pos_table)


def kernel(input_ids, token_table, pos_table):
    ids_flat = input_ids.reshape(_N).astype(jnp.int32)
    out = _emb(ids_flat, token_table, pos_table)
    return out.reshape(_BATCH, _SEQ, _D)


# speedup vs baseline: 2.4603x; 1.0962x over previous
import functools

import jax
import jax.numpy as jnp
from jax import lax
from jax.experimental import pallas as pl
from jax.experimental.pallas import tpu as pltpu
from jax.experimental.pallas import tpu_sc as plsc

_VOCAB = 100000
_D = 768
_BATCH = 4
_SEQ = 2048
_N = _BATCH * _SEQ
_NC = 2
_NS = 16
_NW = _NC * _NS
_POS_W = _SEQ // _NW
_PSUB = 16
_NSUB = _POS_W // _PSUB
_ROWS = _BATCH * _PSUB
_LANES = 16
_VECS = _D // _LANES


def _emb_body(ids_hbm, tok_hbm, pos_hbm, out_hbm,
              ids_stage, idx_v, tv0, tv1, pv0, pv1,
              s_in0, s_in1, s_out0, s_out1):
    wid = lax.axis_index("s") * _NC + lax.axis_index("c")
    pos0 = wid * _POS_W

    tv = (tv0, tv1)
    pv = (pv0, pv1)
    s_in = (s_in0, s_in1)
    s_out = (s_out0, s_out1)

    stage_h = [
        pltpu.async_copy(ids_hbm.at[pl.ds(bat * _SEQ + pos0, _POS_W)],
                         ids_stage.at[pl.ds(bat * _POS_W, _POS_W)], s_in0)
        for bat in range(_BATCH)
    ]
    for hnd in stage_h:
        hnd.wait()
    for h in range(_NSUB):
        for bat in range(_BATCH):
            idx_v[pl.ds((h * _BATCH + bat) * _PSUB, _PSUB)] = (
                ids_stage[pl.ds(bat * _POS_W + h * _PSUB, _PSUB)])

    def fire(h, b):
        g = pltpu.async_copy(tok_hbm.at[idx_v.at[pl.ds(h * _ROWS, _ROWS)]],
                             tv[b], s_in[b])
        p = pltpu.async_copy(pos_hbm.at[pl.ds(pos0 + h * _PSUB, _PSUB)],
                             pv[b], s_in[b])
        return (g, p)

    in_h = [None, None]
    out_h = [[], []]
    in_h[0] = fire(0, 0)

    for h in range(_NSUB):
        b = h & 1
        nb = 1 - b
        if h + 1 < _NSUB:
            for hnd in out_h[nb]:
                hnd.wait()
            out_h[nb] = []
            in_h[nb] = fire(h + 1, nb)
        in_h[b][0].wait()
        in_h[b][1].wait()

        tvb, pvb = tv[b], pv[b]

        @plsc.parallel_loop(0, _PSUB * _VECS, 1, unroll=8)
        def _add(k):
            p = k & (_PSUB - 1)
            j = k >> 4
            s = pl.ds(j * _LANES, _LANES)
            pvec = pvb[p, s]
            for bat in range(_BATCH):
                r = bat * _PSUB + p
                tvb[r, s] = tvb[r, s] + pvec

        for bat in range(_BATCH):
            out_h[b].append(pltpu.async_copy(
                tvb.at[pl.ds(bat * _PSUB, _PSUB)],
                out_hbm.at[pl.ds(bat * _SEQ + pos0 + h * _PSUB, _PSUB)],
                s_out[b]))

    for hnds in out_h:
        for hnd in hnds:
            hnd.wait()


@jax.jit
def _emb(ids_flat, token_table, pos_table):
    mesh = plsc.VectorSubcoreMesh(core_axis_name="c", subcore_axis_name="s")
    run = functools.partial(
        pl.kernel,
        mesh=mesh,
        out_type=jax.ShapeDtypeStruct((_N, _D), jnp.float32),
        scratch_types=[
            pltpu.VMEM((_BATCH * _POS_W,), jnp.int32),
            pltpu.VMEM((_BATCH * _POS_W,), jnp.int32),
            pltpu.VMEM((_ROWS, _D), jnp.float32),
            pltpu.VMEM((_ROWS, _D), jnp.float32),
            pltpu.VMEM((_PSUB, _D), jnp.float32),
            pltpu.VMEM((_PSUB, _D), jnp.float32),
            pltpu.SemaphoreType.DMA,
            pltpu.SemaphoreType.DMA,
            pltpu.SemaphoreType.DMA,
            pltpu.SemaphoreType.DMA,
        ],
    )(_emb_body)
    return run(ids_flat, token_table, pos_table)


def kernel(input_ids, token_table, pos_table):
    ids_flat = input_ids.reshape(_N).astype(jnp.int32)
    out = _emb(ids_flat, token_table, pos_table)
    return out.reshape(_BATCH, _SEQ, _D)


# speedup vs baseline: 2.4856x; 1.0103x over previous
import functools

import jax
import jax.numpy as jnp
from jax import lax
from jax.experimental import pallas as pl
from jax.experimental.pallas import tpu as pltpu
from jax.experimental.pallas import tpu_sc as plsc

_VOCAB = 100000
_D = 768
_BATCH = 4
_SEQ = 2048
_N = _BATCH * _SEQ
_NC = 2
_NS = 16
_NW = _NC * _NS
_POS_W = _SEQ // _NW
_PSUB = 8
_NSUB = _POS_W // _PSUB
_ROWS = _BATCH * _PSUB
_NBUF = 3
_LANES = 16
_VECS = _D // _LANES


def _emb_body(ids_hbm, tok_hbm, pos_hbm, out_hbm,
              ids_stage, idx_v,
              tv0, tv1, tv2, pv0, pv1, pv2,
              si0, si1, si2, so0, so1, so2):
    wid = lax.axis_index("s") * _NC + lax.axis_index("c")
    pos0 = wid * _POS_W

    tv = (tv0, tv1, tv2)
    pv = (pv0, pv1, pv2)
    s_in = (si0, si1, si2)
    s_out = (so0, so1, so2)

    stage_h = [
        pltpu.async_copy(ids_hbm.at[pl.ds(bat * _SEQ + pos0, _POS_W)],
                         ids_stage.at[pl.ds(bat * _POS_W, _POS_W)], si0)
        for bat in range(_BATCH)
    ]
    for hnd in stage_h:
        hnd.wait()
    for h in range(_NSUB):
        for bat in range(_BATCH):
            idx_v[pl.ds((h * _BATCH + bat) * _PSUB, _PSUB)] = (
                ids_stage[pl.ds(bat * _POS_W + h * _PSUB, _PSUB)])

    def fire(h, b):
        g = pltpu.async_copy(tok_hbm.at[idx_v.at[pl.ds(h * _ROWS, _ROWS)]],
                             tv[b], s_in[b])
        p = pltpu.async_copy(pos_hbm.at[pl.ds(pos0 + h * _PSUB, _PSUB)],
                             pv[b], s_in[b])
        return (g, p)

    in_h = [None] * _NBUF
    out_h = [[] for _ in range(_NBUF)]
    for h in range(_NBUF - 1):
        in_h[h] = fire(h, h)

    for h in range(_NSUB):
        b = h % _NBUF
        nxt = h + _NBUF - 1
        if nxt < _NSUB:
            nb = nxt % _NBUF
            for hnd in out_h[nb]:
                hnd.wait()
            out_h[nb] = []
            in_h[nb] = fire(nxt, nb)
        in_h[b][0].wait()
        in_h[b][1].wait()

        tvb, pvb = tv[b], pv[b]

        @plsc.parallel_loop(0, _PSUB * _VECS, 1, unroll=8)
        def _add(k):
            p = k & (_PSUB - 1)
            j = k >> 3
            s = pl.ds(j * _LANES, _LANES)
            pvec = pvb[p, s]
            for bat in range(_BATCH):
                r = bat * _PSUB + p
                tvb[r, s] = tvb[r, s] + pvec

        for bat in range(_BATCH):
            out_h[b].append(pltpu.async_copy(
                tvb.at[pl.ds(bat * _PSUB, _PSUB)],
                out_hbm.at[pl.ds(bat * _SEQ + pos0 + h * _PSUB, _PSUB)],
                s_out[b]))

    for hnds in out_h:
        for hnd in hnds:
            hnd.wait()


@jax.jit
def _emb(ids_flat, token_table, pos_table):
    mesh = plsc.VectorSubcoreMesh(core_axis_name="c", subcore_axis_name="s")
    run = functools.partial(
        pl.kernel,
        mesh=mesh,
        out_type=jax.ShapeDtypeStruct((_N, _D), jnp.float32),
        scratch_types=[
            pltpu.VMEM((_BATCH * _POS_W,), jnp.int32),
            pltpu.VMEM((_BATCH * _POS_W,), jnp.int32),
            pltpu.VMEM((_ROWS, _D), jnp.float32),
            pltpu.VMEM((_ROWS, _D), jnp.float32),
            pltpu.VMEM((_ROWS, _D), jnp.float32),
            pltpu.VMEM((_PSUB, _D), jnp.float32),
            pltpu.VMEM((_PSUB, _D), jnp.float32),
            pltpu.VMEM((_PSUB, _D), jnp.float32),
            pltpu.SemaphoreType.DMA,
            pltpu.SemaphoreType.DMA,
            pltpu.SemaphoreType.DMA,
            pltpu.SemaphoreType.DMA,
            pltpu.SemaphoreType.DMA,
            pltpu.SemaphoreType.DMA,
        ],
    )(_emb_body)
    return run(ids_flat, token_table, pos_table)


def kernel(input_ids, token_table, pos_table):
    ids_flat = input_ids.reshape(_N).astype(jnp.int32)
    out = _emb(ids_flat, token_table, pos_table)
    return out.reshape(_BATCH, _SEQ, _D)
